# baseline (speedup 1.0000x reference)
import jax
import jax.numpy as jnp
from jax.experimental import pallas as pl
from jax.experimental.pallas import tpu as pltpu

_RATE = 0.7
_B = 16384
_C = 1000
_BLK = 512
_G = _B // _BLK
_K = min(_B, int(_B * _RATE))


def _f32_to_ordkey(x):
    b = jax.lax.bitcast_convert_type(x, jnp.uint32)
    neg = (b >> 31) == 1
    return jnp.where(neg, ~b, b | jnp.uint32(0x80000000))


def _ordkey_to_f32(k):
    msb = (k >> 31) == 1
    b = jnp.where(msb, k & jnp.uint32(0x7FFFFFFF), ~k)
    return jax.lax.bitcast_convert_type(b, jnp.float32)


def _ohem_body(pred_ref, tgt_ref, out_ref, nll_ref):
    i = pl.program_id(0)
    x = pred_ref[...]
    t = tgt_ref[0, 0, :]
    m = jnp.max(x, axis=1, keepdims=True)
    s = jnp.sum(jnp.exp(x - m), axis=1, keepdims=True)
    lse = m[:, 0] + jnp.log(s[:, 0])
    cols = jax.lax.broadcasted_iota(jnp.int32, (_BLK, _C), 1)
    pick = jnp.sum(jnp.where(cols == t[:, None], x, 0.0), axis=1)
    nll = jnp.where(t < 0, 0.0, lse - pick)
    nll_ref[pl.ds(i, 1), :] = nll[None, :]

    @pl.when(i == _G - 1)
    def _topk():
        vals = nll_ref[...]
        keys = _f32_to_ordkey(vals)

        def bit_step(j, prefix):
            cand = prefix | (jnp.uint32(1) << (jnp.uint32(31) - j.astype(jnp.uint32)))
            cnt = jnp.sum((keys >= cand).astype(jnp.int32))
            return jnp.where(cnt >= _K, cand, prefix)

        kth = jax.lax.fori_loop(0, 32, bit_step, jnp.uint32(0))
        gt = keys > kth
        cnt_gt = jnp.sum(gt.astype(jnp.int32))
        sum_gt = jnp.sum(jnp.where(gt, vals, 0.0))
        kth_val = _ordkey_to_f32(kth)
        total = sum_gt + (_K - cnt_gt).astype(jnp.float32) * kth_val
        out_ref[0, 0] = total / jnp.float32(_K)


def kernel(pred, target, interpret=False):
    tgt3 = target.astype(jnp.int32).reshape(_G, 1, _BLK)
    out = pl.pallas_call(
        _ohem_body,
        grid=(_G,),
        in_specs=[
            pl.BlockSpec((_BLK, _C), lambda i: (i, 0)),
            pl.BlockSpec((1, 1, _BLK), lambda i: (i, 0, 0)),
        ],
        out_specs=pl.BlockSpec(memory_space=pltpu.SMEM),
        out_shape=jax.ShapeDtypeStruct((1, 1), jnp.float32),
        scratch_shapes=[pltpu.VMEM((_G, _BLK), jnp.float32)],
        interpret=interpret,
    )(pred, tgt3)
    return out[0, 0]


# speedup vs baseline: 1.1251x; 1.1251x over previous
import jax
import jax.numpy as jnp
from jax.experimental import pallas as pl
from jax.experimental.pallas import tpu as pltpu

_RATE = 0.7
_B = 16384
_C = 1000
_BLK = 2048
_G = _B // _BLK
_K = min(_B, int(_B * _RATE))


def _f32_to_ordkey(x):
    b = jax.lax.bitcast_convert_type(x, jnp.uint32)
    neg = (b >> 31) == 1
    return jnp.where(neg, ~b, b | jnp.uint32(0x80000000))


def _ordkey_to_f32(k):
    msb = (k >> 31) == 1
    b = jnp.where(msb, k & jnp.uint32(0x7FFFFFFF), ~k)
    return jax.lax.bitcast_convert_type(b, jnp.float32)


def _ohem_body(pred_ref, tgt_ref, out_ref, nll_ref):
    i = pl.program_id(0)
    x = pred_ref[...]
    t = tgt_ref[0, 0, :]
    m = jnp.max(x, axis=1, keepdims=True)
    s = jnp.sum(jnp.exp(x - m), axis=1, keepdims=True)
    lse = m[:, 0] + jnp.log(s[:, 0])
    cols = jax.lax.broadcasted_iota(jnp.int32, (_BLK, _C), 1)
    pick = jnp.sum(jnp.where(cols == t[:, None], x, 0.0), axis=1)
    nll = jnp.where(t < 0, 0.0, lse - pick)
    nll_ref[pl.ds(i, 1), :] = nll[None, :]

    @pl.when(i == _G - 1)
    def _topk():
        vals = nll_ref[...]
        keys = _f32_to_ordkey(vals)

        def bit_step(j, prefix):
            cand = prefix | (jnp.uint32(1) << (jnp.uint32(31) - j.astype(jnp.uint32)))
            cnt = jnp.sum((keys >= cand).astype(jnp.int32))
            return jnp.where(cnt >= _K, cand, prefix)

        kth = jax.lax.fori_loop(0, 32, bit_step, jnp.uint32(0))
        gt = keys > kth
        cnt_gt = jnp.sum(gt.astype(jnp.int32))
        sum_gt = jnp.sum(jnp.where(gt, vals, 0.0))
        kth_val = _ordkey_to_f32(kth)
        total = sum_gt + (_K - cnt_gt).astype(jnp.float32) * kth_val
        out_ref[0, 0] = total / jnp.float32(_K)


def kernel(pred, target, interpret=False):
    tgt3 = target.astype(jnp.int32).reshape(_G, 1, _BLK)
    out = pl.pallas_call(
        _ohem_body,
        grid=(_G,),
        in_specs=[
            pl.BlockSpec((_BLK, _C), lambda i: (i, 0)),
            pl.BlockSpec((1, 1, _BLK), lambda i: (i, 0, 0)),
        ],
        out_specs=pl.BlockSpec(memory_space=pltpu.SMEM),
        out_shape=jax.ShapeDtypeStruct((1, 1), jnp.float32),
        scratch_shapes=[pltpu.VMEM((_G, _BLK), jnp.float32)],
        interpret=interpret,
    )(pred, tgt3)
    return out[0, 0]
